# Initial kernel scaffold; baseline (speedup 1.0000x reference)
import functools

import jax
import jax.numpy as jnp
from jax import lax
from jax.experimental import pallas as pl
from jax.experimental.pallas import tpu as pltpu
from jax.experimental.pallas import tpu_sc as plsc

N = 10000
E = 160000
R = 2 * E
C = 128
W = 80
NSC = 2
NSUB = 16

f32 = jnp.float32


def _zero_rows(zbuf, nrow):
    w = zbuf.shape[1]
    z16 = jnp.zeros((16,), f32)

    def body(i, carry):
        for k in range(w // 16):
            zbuf[i, pl.ds(k * 16, 16)] = z16
        return carry

    lax.fori_loop(0, nrow, body, 0)


def _sc_st_body(er2, nid2, nsw2, s_out, t_out, stab, ttab, erbuf, nidb, nswb, zbuf):
    cid = lax.axis_index("c")
    sid = lax.axis_index("s")
    _zero_rows(zbuf, 125)
    for r in range(5):
        dst = pl.ds(sid * 625 + r * 125, 125)
        pltpu.sync_copy(zbuf, stab.at[dst])
        pltpu.sync_copy(zbuf, ttab.at[dst])
    plsc.subcore_barrier()

    rows_per_tile = R // NSUB
    chunks = rows_per_tile // W
    inner = 10
    outer = chunks // inner

    def body(ob, carry):
        crow = sid * chunks + ob * inner
        rrow = sid * rows_per_tile + ob * inner * W
        pltpu.sync_copy(nid2.at[pl.ds(crow, inner)], nidb)
        pltpu.sync_copy(nsw2.at[pl.ds(crow, inner)], nswb)
        pltpu.sync_copy(er2.at[cid, pl.ds(rrow, inner * W)], erbuf)
        for j in range(inner):
            src = erbuf.at[pl.ds(j * W, W)]
            pltpu.sync_copy(src, stab.at[nidb.at[j]], add=True)
            pltpu.sync_copy(src, ttab.at[nswb.at[j]], add=True)
        return carry

    lax.fori_loop(0, outer, body, 0)
    plsc.subcore_barrier()
    mine = pl.ds(sid * 625, 625)
    pltpu.sync_copy(stab.at[mine], s_out.at[cid, mine])
    pltpu.sync_copy(ttab.at[mine], t_out.at[cid, mine])


def _sc_st(er2, nid2, nsw2):
    mesh = plsc.VectorSubcoreMesh(core_axis_name="c", subcore_axis_name="s")
    k = pl.kernel(
        _sc_st_body,
        out_type=(
            jax.ShapeDtypeStruct((NSC, N, C // 2), f32),
            jax.ShapeDtypeStruct((NSC, N, C // 2), f32),
        ),
        mesh=mesh,
        scratch_types=[
            pltpu.VMEM_SHARED((N, C // 2), f32),
            pltpu.VMEM_SHARED((N, C // 2), f32),
            pltpu.VMEM((10 * W, C // 2), f32),
            pltpu.VMEM((10, W), jnp.int32),
            pltpu.VMEM((10, W), jnp.int32),
            pltpu.VMEM((125, C // 2), f32),
        ],
    )
    return k(er2, nid2, nsw2)


def _sc_nbr_body(tbl, nid2, nsw2, o_out, d_out, otab, dtab, gbuf, nidb, nswb,
                 zbuf, zbufd, obuf, with_degree):
    cid = lax.axis_index("c")
    sid = lax.axis_index("s")
    _zero_rows(zbuf, 125)
    for r in range(5):
        dst = pl.ds(sid * 625 + r * 125, 125)
        pltpu.sync_copy(zbuf, otab.at[dst])
    if with_degree:
        _zero_rows(zbufd, 125)
        one16 = jnp.ones((16,), f32)

        def ones_body(i, carry):
            obuf[i, pl.ds(0, 16)] = one16
            return carry

        lax.fori_loop(0, W, ones_body, 0)
        for r in range(5):
            dst = pl.ds(sid * 625 + r * 125, 125)
            pltpu.sync_copy(zbufd, dtab.at[dst])
    plsc.subcore_barrier()

    rows_per_tile = R // (NSC * NSUB)
    chunks = rows_per_tile // W
    inner = 5
    outer = chunks // inner

    def body(ob, carry):
        crow = (cid * NSUB + sid) * chunks + ob * inner
        pltpu.sync_copy(nid2.at[pl.ds(crow, inner)], nidb)
        pltpu.sync_copy(nsw2.at[pl.ds(crow, inner)], nswb)
        for j in range(inner):
            pltpu.sync_copy(tbl.at[nswb.at[j]], gbuf)
            pltpu.sync_copy(gbuf, otab.at[nidb.at[j]], add=True)
            if with_degree:
                pltpu.sync_copy(obuf, dtab.at[nidb.at[j]], add=True)
        return carry

    lax.fori_loop(0, outer, body, 0)
    plsc.subcore_barrier()
    mine = pl.ds(sid * 625, 625)
    pltpu.sync_copy(otab.at[mine], o_out.at[cid, mine])
    if with_degree:
        pltpu.sync_copy(dtab.at[mine], d_out.at[cid, mine])


def _sc_nbr(tbl, nid2, nsw2, with_degree):
    mesh = plsc.VectorSubcoreMesh(core_axis_name="c", subcore_axis_name="s")
    if not with_degree:
        def body2(tbl, nid2, nsw2, o_out, otab, gbuf, nidb, nswb, zbuf):
            _sc_nbr_body(tbl, nid2, nsw2, o_out, None, otab, None, gbuf,
                         nidb, nswb, zbuf, None, None, with_degree=False)

        k = pl.kernel(
            body2,
            out_type=jax.ShapeDtypeStruct((NSC, N, C), f32),
            mesh=mesh,
            scratch_types=[
                pltpu.VMEM_SHARED((N, C), f32),
                pltpu.VMEM((W, C), f32),
                pltpu.VMEM((5, W), jnp.int32),
                pltpu.VMEM((5, W), jnp.int32),
                pltpu.VMEM((125, C), f32),
            ],
        )
        return (k(tbl, nid2, nsw2),)

    def body3(tbl, nid2, nsw2, o_out, d_out, otab, dtab, gbuf, nidb, nswb,
              zbuf, zbufd, obuf):
        _sc_nbr_body(tbl, nid2, nsw2, o_out, d_out, otab, dtab, gbuf, nidb,
                     nswb, zbuf, zbufd, obuf, with_degree=True)

    k = pl.kernel(
        body3,
        out_type=(
            jax.ShapeDtypeStruct((NSC, N, C), f32),
            jax.ShapeDtypeStruct((NSC, N, 16), f32),
        ),
        mesh=mesh,
        scratch_types=[
            pltpu.VMEM_SHARED((N, C), f32),
            pltpu.VMEM_SHARED((N, 16), f32),
            pltpu.VMEM((W, C), f32),
            pltpu.VMEM((5, W), jnp.int32),
            pltpu.VMEM((5, W), jnp.int32),
            pltpu.VMEM((125, C), f32),
            pltpu.VMEM((125, 16), f32),
            pltpu.VMEM((W, 16), f32),
        ],
    )
    return k(tbl, nid2, nsw2)


def _sc_zp_body(f1, f2, nid2, nsw2, zp_out, gbuf, nidb, nswb):
    cid = lax.axis_index("c")
    sid = lax.axis_index("s")
    rows_per_tile = R // (NSC * NSUB)
    chunks = rows_per_tile // W
    inner = 5
    outer = chunks // inner

    def body(ob, carry):
        crow = (cid * NSUB + sid) * chunks + ob * inner
        rrow = crow * W
        pltpu.sync_copy(nid2.at[pl.ds(crow, inner)], nidb)
        pltpu.sync_copy(nsw2.at[pl.ds(crow, inner)], nswb)
        for j in range(inner):
            pltpu.sync_copy(f1.at[nidb.at[j]], gbuf)
            pltpu.sync_copy(f2.at[nswb.at[j]], gbuf, add=True)
            pltpu.sync_copy(gbuf, zp_out.at[pl.ds(rrow + j * W, W)])
        return carry

    lax.fori_loop(0, outer, body, 0)


def _sc_zp(f1, f2, nid2, nsw2):
    mesh = plsc.VectorSubcoreMesh(core_axis_name="c", subcore_axis_name="s")
    k = pl.kernel(
        _sc_zp_body,
        out_type=jax.ShapeDtypeStruct((R, 2 * C), f32),
        mesh=mesh,
        scratch_types=[
            pltpu.VMEM((W, 2 * C), f32),
            pltpu.VMEM((5, W), jnp.int32),
            pltpu.VMEM((5, W), jnp.int32),
        ],
    )
    return k(f1, f2, nid2, nsw2)


def _tc_proj_body(s_ref, t_ref, x_ref, dp_ref, w7_ref, f1_ref, f2_ref, w04_ref):
    i = pl.program_id(0)
    w = w7_ref[...]
    w1234 = w[1] + w[2] + w[3] + w[4]
    w24 = w[2] + w[4]
    w34 = w[3] + w[4]
    w56 = w[5] + w[6]
    s = s_ref[...]
    t = t_ref[...]
    x = x_ref[...]
    d = dp_ref[0, :, 0:1] + dp_ref[1, :, 0:1]
    ds_ = d * s
    dx = d * x
    dot = functools.partial(jnp.dot, preferred_element_type=f32)
    f1_ref[...] = dot(ds_, w1234) + dot(t, w24) + dot(dx, w56)
    f2_ref[...] = dot(s, w34) + dot(x, w[6])

    @pl.when(i == 0)
    def _():
        w04_ref[...] = w[0] + w[4]


def _tc_proj(s, t, x, dp, w7):
    blk = 500
    return pl.pallas_call(
        _tc_proj_body,
        grid=(N // blk,),
        in_specs=[
            pl.BlockSpec((blk, C), lambda i: (i, 0)),
            pl.BlockSpec((blk, C), lambda i: (i, 0)),
            pl.BlockSpec((blk, C), lambda i: (i, 0)),
            pl.BlockSpec((NSC, blk, 16), lambda i: (0, i, 0)),
            pl.BlockSpec((7, C, 2 * C), lambda i: (0, 0, 0)),
        ],
        out_specs=[
            pl.BlockSpec((blk, 2 * C), lambda i: (i, 0)),
            pl.BlockSpec((blk, 2 * C), lambda i: (i, 0)),
            pl.BlockSpec((C, 2 * C), lambda i: (0, 0)),
        ],
        out_shape=[
            jax.ShapeDtypeStruct((N, 2 * C), f32),
            jax.ShapeDtypeStruct((N, 2 * C), f32),
            jax.ShapeDtypeStruct((C, 2 * C), f32),
        ],
    )(s, t, x, dp, w7)


def _tc_node_body(x_ref, s_ref, t_ref, xsp_ref, ssp_ref, dp_ref, w7_ref,
                  gn1_ref, bn1_ref, wn2_ref, gn2_ref, bn2_ref, out_ref):
    x = x_ref[...]
    s = s_ref[...]
    t = t_ref[...]
    xs = xsp_ref[0] + xsp_ref[1]
    ss = ssp_ref[0] + ssp_ref[1]
    d = dp_ref[0, :, 0:1] + dp_ref[1, :, 0:1]
    w = w7_ref[...]
    ds_ = d * s
    d2s = d * ds_
    dt = d * t
    dx = d * x
    d2x = d * dx
    dot = functools.partial(jnp.dot, preferred_element_type=f32)
    z = (dot(x, w[0]) + dot(d2s, w[1] + w[2] + w[3] + w[4])
         + dot(dt, w[2] + w[4]) + dot(ss, w[3] + w[4]) + dot(s, w[4])
         + dot(d2x, w[5] + w[6]) + dot(xs, w[6]))
    m = jnp.mean(z, axis=0, keepdims=True)
    v = jnp.mean((z - m) ** 2, axis=0, keepdims=True)
    h = jnp.maximum(gn1_ref[...] * (z - m) * lax.rsqrt(v + 1e-5) + bn1_ref[...], 0.0)
    z2 = dot(h, wn2_ref[...])
    m2 = jnp.mean(z2, axis=0, keepdims=True)
    v2 = jnp.mean((z2 - m2) ** 2, axis=0, keepdims=True)
    out_ref[...] = jnp.maximum(
        gn2_ref[...] * (z2 - m2) * lax.rsqrt(v2 + 1e-5) + bn2_ref[...], 0.0)


def _tc_node(x, s, t, xsp, ssp, dp, w7, gn1, bn1, wn2, gn2, bn2):
    return pl.pallas_call(
        _tc_node_body,
        out_shape=jax.ShapeDtypeStruct((N, C), f32),
    )(x, s, t, xsp, ssp, dp, w7, gn1.reshape(1, -1), bn1.reshape(1, -1),
      wn2, gn2.reshape(1, -1), bn2.reshape(1, -1))


def _tc_estats_body(er_ref, zp_ref, w04_ref, out_ref):
    i = pl.program_id(0)

    @pl.when(i == 0)
    def _():
        out_ref[...] = jnp.zeros_like(out_ref)

    z = (jnp.dot(er_ref[...], w04_ref[...], preferred_element_type=f32)
         + zp_ref[...])
    out_ref[0:1, :] += jnp.sum(z, axis=0, keepdims=True)
    out_ref[1:2, :] += jnp.sum(z * z, axis=0, keepdims=True)


def _tc_estats(er, zp, w04, blk):
    return pl.pallas_call(
        _tc_estats_body,
        grid=(R // blk,),
        in_specs=[
            pl.BlockSpec((blk, C), lambda i: (i, 0)),
            pl.BlockSpec((blk, 2 * C), lambda i: (i, 0)),
            pl.BlockSpec((C, 2 * C), lambda i: (0, 0)),
        ],
        out_specs=pl.BlockSpec((8, 2 * C), lambda i: (0, 0)),
        out_shape=jax.ShapeDtypeStruct((8, 2 * C), f32),
    )(er, zp, w04)


def _tc_elayer_body(er_ref, zp_ref, w04_ref, st1_ref, ge1_ref, be1_ref,
                    we2_ref, h2_ref, st2_ref):
    i = pl.program_id(0)
    z = (jnp.dot(er_ref[...], w04_ref[...], preferred_element_type=f32)
         + zp_ref[...])
    m = st1_ref[0:1, :] * (1.0 / R)
    v = st1_ref[1:2, :] * (1.0 / R) - m * m
    h = jnp.maximum(ge1_ref[...] * (z - m) * lax.rsqrt(v + 1e-5) + be1_ref[...], 0.0)
    h2 = jnp.dot(h, we2_ref[...], preferred_element_type=f32)
    h2_ref[...] = h2

    @pl.when(i == 0)
    def _():
        st2_ref[...] = jnp.zeros_like(st2_ref)

    st2_ref[0:1, :] += jnp.sum(h2, axis=0, keepdims=True)
    st2_ref[1:2, :] += jnp.sum(h2 * h2, axis=0, keepdims=True)


def _tc_elayer(er, zp, w04, st1, ge1, be1, we2, blk):
    return pl.pallas_call(
        _tc_elayer_body,
        grid=(R // blk,),
        in_specs=[
            pl.BlockSpec((blk, C), lambda i: (i, 0)),
            pl.BlockSpec((blk, 2 * C), lambda i: (i, 0)),
            pl.BlockSpec((C, 2 * C), lambda i: (0, 0)),
            pl.BlockSpec((8, 2 * C), lambda i: (0, 0)),
            pl.BlockSpec((1, 2 * C), lambda i: (0, 0)),
            pl.BlockSpec((1, 2 * C), lambda i: (0, 0)),
            pl.BlockSpec((2 * C, C), lambda i: (0, 0)),
        ],
        out_specs=[
            pl.BlockSpec((blk, C), lambda i: (i, 0)),
            pl.BlockSpec((8, C), lambda i: (0, 0)),
        ],
        out_shape=[
            jax.ShapeDtypeStruct((R, C), f32),
            jax.ShapeDtypeStruct((8, C), f32),
        ],
    )(er, zp, w04, st1, ge1.reshape(1, -1), be1.reshape(1, -1), we2)


def _tc_efinal_body(h2_ref, st2_ref, ge2_ref, be2_ref, out_ref):
    m = st2_ref[0:1, :] * (1.0 / R)
    v = st2_ref[1:2, :] * (1.0 / R) - m * m
    out_ref[...] = jnp.maximum(
        ge2_ref[...] * (h2_ref[...] - m) * lax.rsqrt(v + 1e-5) + be2_ref[...], 0.0)


def _tc_efinal(h2, st2, ge2, be2, blk):
    return pl.pallas_call(
        _tc_efinal_body,
        grid=(R // blk,),
        in_specs=[
            pl.BlockSpec((blk, C), lambda i: (i, 0)),
            pl.BlockSpec((8, C), lambda i: (0, 0)),
            pl.BlockSpec((1, C), lambda i: (0, 0)),
            pl.BlockSpec((1, C), lambda i: (0, 0)),
        ],
        out_specs=pl.BlockSpec((blk, C), lambda i: (i, 0)),
        out_shape=jax.ShapeDtypeStruct((R, C), f32),
    )(h2, st2, ge2.reshape(1, -1), be2.reshape(1, -1))


def kernel(node_rep, edge_rep, edge_index, Wn1, gn1, bn1, Wn2, gn2, bn2,
           We1, ge1, be1, We2, ge2, be2):
    u = edge_index[0]
    v = edge_index[1]
    nid2 = jnp.stack([u, v], axis=1).reshape(R // W, W)
    nsw2 = jnp.stack([v, u], axis=1).reshape(R // W, W)
    er2 = edge_rep.reshape(R, 2, C // 2).transpose(1, 0, 2)
    w7e = We1.reshape(7, C, 2 * C)
    w7n = Wn1.reshape(7, C, 2 * C)

    xsp, dp = _sc_nbr(node_rep, nid2, nsw2, with_degree=True)
    s2, t2 = _sc_st(er2, nid2, nsw2)
    s_full = jnp.concatenate([s2[0], s2[1]], axis=-1)
    t_full = jnp.concatenate([t2[0], t2[1]], axis=-1)
    (ssp,) = _sc_nbr(s_full, nid2, nsw2, with_degree=False)

    f1, f2, w04 = _tc_proj(s_full, t_full, node_rep, dp, w7e)
    zp = _sc_zp(f1, f2, nid2, nsw2)

    node_out = _tc_node(node_rep, s_full, t_full, xsp, ssp, dp, w7n,
                        gn1, bn1, Wn2, gn2, bn2)

    blk = 2000
    st1 = _tc_estats(edge_rep, zp, w04, blk)
    h2, st2 = _tc_elayer(edge_rep, zp, w04, st1, ge1, be1, We2, blk)
    edge_out = _tc_efinal(h2, st2, ge2, be2, blk)
    return (node_out, edge_out)


# speedup vs baseline: 5.3606x; 5.3606x over previous
import functools

import jax
import jax.numpy as jnp
from jax import lax
from jax.experimental import pallas as pl
from jax.experimental.pallas import tpu as pltpu
from jax.experimental.pallas import tpu_sc as plsc

N = 10000
NP = 10240
E = 160000
R = 2 * E
C = 128
W = 125
NCHUNK = R // W
INNER = 8
NSC = 2
NSUB = 16
RPT_N = NP // NSUB

f32 = jnp.float32


def _zero_buf(zbuf):
    rows, w = zbuf.shape
    z16 = jnp.zeros((16,), f32)

    def body(i, carry):
        for k in range(w // 16):
            zbuf[i, pl.ds(k * 16, 16)] = z16
        return carry

    lax.fori_loop(0, rows, body, 0)


def _fill_table(zbuf, tab, sid):
    for r in range(RPT_N // 64):
        pltpu.sync_copy(zbuf, tab.at[pl.ds(sid * RPT_N + r * 64, 64)])


def _sc_seg_body(er3, idx2, o_out, tab, erbuf, idxb, zbuf):
    cid = lax.axis_index("c")
    sid = lax.axis_index("s")
    _zero_buf(zbuf)
    _fill_table(zbuf, tab, sid)
    plsc.subcore_barrier()

    chunks = NCHUNK // (NSC * NSUB)
    outer = chunks // INNER

    def body(ob, carry):
        crow = (cid * NSUB + sid) * chunks + ob * INNER
        pltpu.sync_copy(idx2.at[pl.ds(crow, INNER)], idxb)
        for j in range(INNER):
            pltpu.sync_copy(er3.at[crow + j], erbuf)
            pltpu.sync_copy(erbuf, tab.at[idxb.at[j]], add=True)
        return carry

    lax.fori_loop(0, outer, body, 0)
    plsc.subcore_barrier()
    mine = pl.ds(sid * RPT_N, RPT_N)
    pltpu.sync_copy(tab.at[mine], o_out.at[cid, mine])


def _sc_seg(er3, idx2):
    mesh = plsc.VectorSubcoreMesh(core_axis_name="c", subcore_axis_name="s", num_cores=NSC, num_subcores=NSUB)
    k = pl.kernel(
        _sc_seg_body,
        out_type=jax.ShapeDtypeStruct((NSC, NP, C), f32),
        mesh=mesh,
        scratch_types=[
            pltpu.VMEM_SHARED((NP, C), f32),
            pltpu.VMEM((W, C), f32),
            pltpu.VMEM((INNER, W), jnp.int32),
            pltpu.VMEM((64, C), f32),
        ],
    )
    return k(er3, idx2)


def _sc_deg_body(idx2, d_out, dtab, obuf, idxb, zbuf):
    cid = lax.axis_index("c")
    sid = lax.axis_index("s")
    _zero_buf(zbuf)
    _fill_table(zbuf, dtab, sid)
    one16 = jnp.ones((16,), f32)

    def ones_body(i, carry):
        for k in range(C // 16):
            obuf[i, pl.ds(16 * k, 16)] = one16
        return carry

    lax.fori_loop(0, W, ones_body, 0)
    plsc.subcore_barrier()

    chunks = NCHUNK // (NSC * NSUB)
    outer = chunks // INNER

    def body(ob, carry):
        crow = (cid * NSUB + sid) * chunks + ob * INNER
        pltpu.sync_copy(idx2.at[pl.ds(crow, INNER)], idxb)
        for j in range(INNER):
            pltpu.sync_copy(obuf, dtab.at[idxb.at[j]], add=True)
        return carry

    lax.fori_loop(0, outer, body, 0)
    plsc.subcore_barrier()
    mine = pl.ds(sid * RPT_N, RPT_N)
    pltpu.sync_copy(dtab.at[mine], d_out.at[cid, mine])


def _sc_deg(idx2):
    mesh = plsc.VectorSubcoreMesh(core_axis_name="c", subcore_axis_name="s", num_cores=NSC, num_subcores=NSUB)
    k = pl.kernel(
        _sc_deg_body,
        out_type=jax.ShapeDtypeStruct((NSC, NP, C), f32),
        mesh=mesh,
        scratch_types=[
            pltpu.VMEM_SHARED((NP, C), f32),
            pltpu.VMEM((W, C), f32),
            pltpu.VMEM((INNER, W), jnp.int32),
            pltpu.VMEM((64, C), f32),
        ],
    )
    return k(idx2)


def _sc_nbr_body(tbl, nid2, nsw2, o_out, otab, gbuf, nidb, nswb, zbuf):
    cid = lax.axis_index("c")
    sid = lax.axis_index("s")
    _zero_buf(zbuf)
    _fill_table(zbuf, otab, sid)
    plsc.subcore_barrier()

    chunks = NCHUNK // (NSC * NSUB)
    outer = chunks // INNER

    def body(ob, carry):
        crow = (cid * NSUB + sid) * chunks + ob * INNER
        pltpu.sync_copy(nid2.at[pl.ds(crow, INNER)], nidb)
        pltpu.sync_copy(nsw2.at[pl.ds(crow, INNER)], nswb)
        for j in range(INNER):
            pltpu.sync_copy(tbl.at[nswb.at[j]], gbuf)
            pltpu.sync_copy(gbuf, otab.at[nidb.at[j]], add=True)
        return carry

    lax.fori_loop(0, outer, body, 0)
    plsc.subcore_barrier()
    mine = pl.ds(sid * RPT_N, RPT_N)
    pltpu.sync_copy(otab.at[mine], o_out.at[cid, mine])


def _sc_nbr(tbl, nid2, nsw2):
    mesh = plsc.VectorSubcoreMesh(core_axis_name="c", subcore_axis_name="s", num_cores=NSC, num_subcores=NSUB)
    k = pl.kernel(
        _sc_nbr_body,
        out_type=jax.ShapeDtypeStruct((NSC, NP, C), f32),
        mesh=mesh,
        scratch_types=[
            pltpu.VMEM_SHARED((NP, C), f32),
            pltpu.VMEM((W, C), f32),
            pltpu.VMEM((INNER, W), jnp.int32),
            pltpu.VMEM((INNER, W), jnp.int32),
            pltpu.VMEM((64, C), f32),
        ],
    )
    return k(tbl, nid2, nsw2)


def _sc_zp_body(f1a, f1b, f2a, f2b, nid2, nsw2, zpa_out, zpb_out,
                gbuf, nidb, nswb):
    cid = lax.axis_index("c")
    sid = lax.axis_index("s")
    chunks = NCHUNK // (NSC * NSUB)
    outer = chunks // INNER

    def body(ob, carry):
        crow = (cid * NSUB + sid) * chunks + ob * INNER
        pltpu.sync_copy(nid2.at[pl.ds(crow, INNER)], nidb)
        pltpu.sync_copy(nsw2.at[pl.ds(crow, INNER)], nswb)
        for j in range(INNER):
            pltpu.sync_copy(f1a.at[nidb.at[j]], gbuf)
            pltpu.sync_copy(f2a.at[nswb.at[j]], gbuf, add=True)
            pltpu.sync_copy(gbuf, zpa_out.at[crow + j])
            pltpu.sync_copy(f1b.at[nidb.at[j]], gbuf)
            pltpu.sync_copy(f2b.at[nswb.at[j]], gbuf, add=True)
            pltpu.sync_copy(gbuf, zpb_out.at[crow + j])
        return carry

    lax.fori_loop(0, outer, body, 0)


def _sc_zp(f1a, f1b, f2a, f2b, nid2, nsw2):
    mesh = plsc.VectorSubcoreMesh(core_axis_name="c", subcore_axis_name="s", num_cores=NSC, num_subcores=NSUB)
    k = pl.kernel(
        _sc_zp_body,
        out_type=(
            jax.ShapeDtypeStruct((NCHUNK, W, C), f32),
            jax.ShapeDtypeStruct((NCHUNK, W, C), f32),
        ),
        mesh=mesh,
        scratch_types=[
            pltpu.VMEM((W, C), f32),
            pltpu.VMEM((INNER, W), jnp.int32),
            pltpu.VMEM((INNER, W), jnp.int32),
        ],
    )
    return k(f1a, f1b, f2a, f2b, nid2, nsw2)


def _tc_proj_body(sp_ref, tp_ref, x_ref, dp_ref, w7_ref,
                  f1a_ref, f1b_ref, f2a_ref, f2b_ref, w04_ref, s_ref, t_ref):
    i = pl.program_id(0)
    w = w7_ref[...]
    w1234 = w[1] + w[2] + w[3] + w[4]
    w24 = w[2] + w[4]
    w34 = w[3] + w[4]
    w56 = w[5] + w[6]
    s = sp_ref[0] + sp_ref[1]
    t = tp_ref[0] + tp_ref[1]
    x = x_ref[...]
    d = dp_ref[0, :, 0:1] + dp_ref[1, :, 0:1]
    s_ref[...] = s
    t_ref[...] = t
    ds_ = d * s
    dx = d * x
    dot = functools.partial(jnp.dot, preferred_element_type=f32)
    f1 = dot(ds_, w1234) + dot(t, w24) + dot(dx, w56)
    f2 = dot(s, w34) + dot(x, w[6])
    f1a_ref[...] = f1[:, :C]
    f1b_ref[...] = f1[:, C:]
    f2a_ref[...] = f2[:, :C]
    f2b_ref[...] = f2[:, C:]

    @pl.when(i == 0)
    def _():
        w04_ref[...] = w[0] + w[4]


def _tc_proj(sp, tp, x, dp, w7):
    blk = 400
    return pl.pallas_call(
        _tc_proj_body,
        grid=(N // blk,),
        in_specs=[
            pl.BlockSpec((NSC, blk, C), lambda i: (0, i, 0)),
            pl.BlockSpec((NSC, blk, C), lambda i: (0, i, 0)),
            pl.BlockSpec((blk, C), lambda i: (i, 0)),
            pl.BlockSpec((NSC, blk, C), lambda i: (0, i, 0)),
            pl.BlockSpec((7, C, 2 * C), lambda i: (0, 0, 0)),
        ],
        out_specs=[
            pl.BlockSpec((blk, C), lambda i: (i, 0)),
            pl.BlockSpec((blk, C), lambda i: (i, 0)),
            pl.BlockSpec((blk, C), lambda i: (i, 0)),
            pl.BlockSpec((blk, C), lambda i: (i, 0)),
            pl.BlockSpec((C, 2 * C), lambda i: (0, 0)),
            pl.BlockSpec((blk, C), lambda i: (i, 0)),
            pl.BlockSpec((blk, C), lambda i: (i, 0)),
        ],
        out_shape=[
            jax.ShapeDtypeStruct((N, C), f32),
            jax.ShapeDtypeStruct((N, C), f32),
            jax.ShapeDtypeStruct((N, C), f32),
            jax.ShapeDtypeStruct((N, C), f32),
            jax.ShapeDtypeStruct((C, 2 * C), f32),
            jax.ShapeDtypeStruct((N, C), f32),
            jax.ShapeDtypeStruct((N, C), f32),
        ],
    )(sp, tp, x, dp, w7)


def _tc_nodez_body(x_ref, s_ref, t_ref, xsp_ref, ssp_ref, dp_ref, w7_ref,
                   z_ref, st_ref):
    i = pl.program_id(0)
    x = x_ref[...]
    s = s_ref[...]
    t = t_ref[...]
    xs = xsp_ref[0] + xsp_ref[1]
    ss = ssp_ref[0] + ssp_ref[1]
    d = dp_ref[0, :, 0:1] + dp_ref[1, :, 0:1]
    w = w7_ref[...]
    ds_ = d * s
    d2s = d * ds_
    dt = d * t
    dx = d * x
    d2x = d * dx
    dot = functools.partial(jnp.dot, preferred_element_type=f32)
    z = (dot(x, w[0]) + dot(d2s, w[1] + w[2] + w[3] + w[4])
         + dot(dt, w[2] + w[4]) + dot(ss, w[3] + w[4]) + dot(s, w[4])
         + dot(d2x, w[5] + w[6]) + dot(xs, w[6]))
    z_ref[...] = z

    @pl.when(i == 0)
    def _():
        st_ref[...] = jnp.zeros_like(st_ref)

    st_ref[0:1, :] += jnp.sum(z, axis=0, keepdims=True)
    st_ref[1:2, :] += jnp.sum(z * z, axis=0, keepdims=True)


def _tc_nodez(x, s, t, xsp, ssp, dp, w7):
    blk = 2000
    return pl.pallas_call(
        _tc_nodez_body,
        grid=(N // blk,),
        in_specs=[
            pl.BlockSpec((blk, C), lambda i: (i, 0)),
            pl.BlockSpec((blk, C), lambda i: (i, 0)),
            pl.BlockSpec((blk, C), lambda i: (i, 0)),
            pl.BlockSpec((NSC, blk, C), lambda i: (0, i, 0)),
            pl.BlockSpec((NSC, blk, C), lambda i: (0, i, 0)),
            pl.BlockSpec((NSC, blk, C), lambda i: (0, i, 0)),
            pl.BlockSpec((7, C, 2 * C), lambda i: (0, 0, 0)),
        ],
        out_specs=[
            pl.BlockSpec((blk, 2 * C), lambda i: (i, 0)),
            pl.BlockSpec((8, 2 * C), lambda i: (0, 0)),
        ],
        out_shape=[
            jax.ShapeDtypeStruct((N, 2 * C), f32),
            jax.ShapeDtypeStruct((8, 2 * C), f32),
        ],
    )(x, s, t, xsp, ssp, dp, w7)


def _tc_norm_mm_body(nrows, y_ref, st_ref, g_ref, b_ref, w2_ref, y2_ref, st2_ref):
    i = pl.program_id(0)
    m = st_ref[0:1, :] * (1.0 / nrows)
    v = st_ref[1:2, :] * (1.0 / nrows) - m * m
    h = jnp.maximum(
        g_ref[...] * (y_ref[...] - m) * lax.rsqrt(v + 1e-5) + b_ref[...], 0.0)
    y2 = jnp.dot(h, w2_ref[...], preferred_element_type=f32)
    y2_ref[...] = y2

    @pl.when(i == 0)
    def _():
        st2_ref[...] = jnp.zeros_like(st2_ref)

    st2_ref[0:1, :] += jnp.sum(y2, axis=0, keepdims=True)
    st2_ref[1:2, :] += jnp.sum(y2 * y2, axis=0, keepdims=True)


def _tc_norm_mm(y, st, g, b, w2, blk):
    nrows, cin = y.shape
    cout = w2.shape[1]
    return pl.pallas_call(
        functools.partial(_tc_norm_mm_body, nrows),
        grid=(nrows // blk,),
        in_specs=[
            pl.BlockSpec((blk, cin), lambda i: (i, 0)),
            pl.BlockSpec((8, cin), lambda i: (0, 0)),
            pl.BlockSpec((1, cin), lambda i: (0, 0)),
            pl.BlockSpec((1, cin), lambda i: (0, 0)),
            pl.BlockSpec((cin, cout), lambda i: (0, 0)),
        ],
        out_specs=[
            pl.BlockSpec((blk, cout), lambda i: (i, 0)),
            pl.BlockSpec((8, cout), lambda i: (0, 0)),
        ],
        out_shape=[
            jax.ShapeDtypeStruct((nrows, cout), f32),
            jax.ShapeDtypeStruct((8, cout), f32),
        ],
    )(y, st, g.reshape(1, -1), b.reshape(1, -1), w2)


def _tc_norm_body(nrows, y_ref, st_ref, g_ref, b_ref, out_ref):
    m = st_ref[0:1, :] * (1.0 / nrows)
    v = st_ref[1:2, :] * (1.0 / nrows) - m * m
    out_ref[...] = jnp.maximum(
        g_ref[...] * (y_ref[...] - m) * lax.rsqrt(v + 1e-5) + b_ref[...], 0.0)


def _tc_norm(y, st, g, b, blk):
    nrows, cin = y.shape
    return pl.pallas_call(
        functools.partial(_tc_norm_body, nrows),
        grid=(nrows // blk,),
        in_specs=[
            pl.BlockSpec((blk, cin), lambda i: (i, 0)),
            pl.BlockSpec((8, cin), lambda i: (0, 0)),
            pl.BlockSpec((1, cin), lambda i: (0, 0)),
            pl.BlockSpec((1, cin), lambda i: (0, 0)),
        ],
        out_specs=pl.BlockSpec((blk, cin), lambda i: (i, 0)),
        out_shape=jax.ShapeDtypeStruct((nrows, cin), f32),
    )(y, st, g.reshape(1, -1), b.reshape(1, -1))


def _tc_estats_body(er_ref, zpa_ref, zpb_ref, w04_ref, out_ref):
    i = pl.program_id(0)

    @pl.when(i == 0)
    def _():
        out_ref[...] = jnp.zeros_like(out_ref)

    zp = jnp.concatenate([zpa_ref[...], zpb_ref[...]], axis=-1)
    z = (jnp.dot(er_ref[...], w04_ref[...], preferred_element_type=f32)
         + zp)
    out_ref[0:1, :] += jnp.sum(z, axis=0, keepdims=True)
    out_ref[1:2, :] += jnp.sum(z * z, axis=0, keepdims=True)


def _tc_estats(er, zpa, zpb, w04, blk):
    return pl.pallas_call(
        _tc_estats_body,
        grid=(R // blk,),
        in_specs=[
            pl.BlockSpec((blk, C), lambda i: (i, 0)),
            pl.BlockSpec((blk, C), lambda i: (i, 0)),
            pl.BlockSpec((blk, C), lambda i: (i, 0)),
            pl.BlockSpec((C, 2 * C), lambda i: (0, 0)),
        ],
        out_specs=pl.BlockSpec((8, 2 * C), lambda i: (0, 0)),
        out_shape=jax.ShapeDtypeStruct((8, 2 * C), f32),
    )(er, zpa, zpb, w04)


def _tc_elayer_body(er_ref, zpa_ref, zpb_ref, w04_ref, st1_ref, ge1_ref,
                    be1_ref, we2_ref, h2_ref, st2_ref):
    i = pl.program_id(0)
    zp = jnp.concatenate([zpa_ref[...], zpb_ref[...]], axis=-1)
    z = (jnp.dot(er_ref[...], w04_ref[...], preferred_element_type=f32)
         + zp)
    m = st1_ref[0:1, :] * (1.0 / R)
    v = st1_ref[1:2, :] * (1.0 / R) - m * m
    h = jnp.maximum(ge1_ref[...] * (z - m) * lax.rsqrt(v + 1e-5) + be1_ref[...], 0.0)
    h2 = jnp.dot(h, we2_ref[...], preferred_element_type=f32)
    h2_ref[...] = h2

    @pl.when(i == 0)
    def _():
        st2_ref[...] = jnp.zeros_like(st2_ref)

    st2_ref[0:1, :] += jnp.sum(h2, axis=0, keepdims=True)
    st2_ref[1:2, :] += jnp.sum(h2 * h2, axis=0, keepdims=True)


def _tc_elayer(er, zpa, zpb, w04, st1, ge1, be1, we2, blk):
    return pl.pallas_call(
        _tc_elayer_body,
        grid=(R // blk,),
        in_specs=[
            pl.BlockSpec((blk, C), lambda i: (i, 0)),
            pl.BlockSpec((blk, C), lambda i: (i, 0)),
            pl.BlockSpec((blk, C), lambda i: (i, 0)),
            pl.BlockSpec((C, 2 * C), lambda i: (0, 0)),
            pl.BlockSpec((8, 2 * C), lambda i: (0, 0)),
            pl.BlockSpec((1, 2 * C), lambda i: (0, 0)),
            pl.BlockSpec((1, 2 * C), lambda i: (0, 0)),
            pl.BlockSpec((2 * C, C), lambda i: (0, 0)),
        ],
        out_specs=[
            pl.BlockSpec((blk, C), lambda i: (i, 0)),
            pl.BlockSpec((8, C), lambda i: (0, 0)),
        ],
        out_shape=[
            jax.ShapeDtypeStruct((R, C), f32),
            jax.ShapeDtypeStruct((8, C), f32),
        ],
    )(er, zpa, zpb, w04, st1, ge1.reshape(1, -1), be1.reshape(1, -1), we2)


def kernel(node_rep, edge_rep, edge_index, Wn1, gn1, bn1, Wn2, gn2, bn2,
           We1, ge1, be1, We2, ge2, be2):
    u = edge_index[0]
    v = edge_index[1]
    nid2 = jnp.stack([u, v], axis=1).reshape(NCHUNK, W)
    nsw2 = jnp.stack([v, u], axis=1).reshape(NCHUNK, W)
    er3 = edge_rep.reshape(NCHUNK, W, C)
    w7e = We1.reshape(7, C, 2 * C)
    w7n = Wn1.reshape(7, C, 2 * C)

    dp = _sc_deg(nid2)
    xsp = _sc_nbr(node_rep, nid2, nsw2)
    sp = _sc_seg(er3, nid2)
    tp = _sc_seg(er3, nsw2)

    dpn = dp[:, :N]
    f1a, f1b, f2a, f2b, w04, s_sum, t_sum = _tc_proj(sp[:, :N], tp[:, :N],
                                                     node_rep, dpn, w7e)
    ssp = _sc_nbr(s_sum, nid2, nsw2)
    zpa3, zpb3 = _sc_zp(f1a, f1b, f2a, f2b, nid2, nsw2)
    zpa = zpa3.reshape(R, C)
    zpb = zpb3.reshape(R, C)

    zn, stn1 = _tc_nodez(node_rep, s_sum, t_sum, xsp[:, :N], ssp[:, :N],
                         dpn, w7n)
    zn2, stn2 = _tc_norm_mm(zn, stn1, gn1, bn1, Wn2, 2000)
    node_out = _tc_norm(zn2, stn2, gn2, bn2, 2000)

    blk = 2000
    st1 = _tc_estats(edge_rep, zpa, zpb, w04, blk)
    h2, st2 = _tc_elayer(edge_rep, zpa, zpb, w04, st1, ge1, be1, We2, blk)
    edge_out = _tc_norm(h2, st2, ge2, be2, blk)
    return (node_out, edge_out)


# speedup vs baseline: 6.3317x; 1.1812x over previous
import functools

import jax
import jax.numpy as jnp
from jax import lax
from jax.experimental import pallas as pl
from jax.experimental.pallas import tpu as pltpu
from jax.experimental.pallas import tpu_sc as plsc

N = 10000
NP = 10240
E = 160000
R = 2 * E
C = 128
W = 125
NCHUNK = R // W
INNER = 8
NSC = 2
NSUB = 16
RPT_N = NP // NSUB

f32 = jnp.float32


def _zero_buf(zbuf):
    rows, w = zbuf.shape
    z16 = jnp.zeros((16,), f32)

    def body(i, carry):
        for k in range(w // 16):
            zbuf[i, pl.ds(k * 16, 16)] = z16
        return carry

    lax.fori_loop(0, rows, body, 0)


def _fill_table(zbuf, tab, sid):
    for r in range(RPT_N // 64):
        pltpu.sync_copy(zbuf, tab.at[pl.ds(sid * RPT_N + r * 64, 64)])


def _sc_seg_body(er3, idx2, o_out, tab, eb0, eb1, idxb, zbuf, sld, sst):
    cid = lax.axis_index("c")
    sid = lax.axis_index("s")
    _zero_buf(zbuf)
    _fill_table(zbuf, tab, sid)
    plsc.subcore_barrier()

    chunks = NCHUNK // (NSC * NSUB)
    outer = chunks // INNER
    EB = (eb0, eb1)

    def body(ob, carry):
        crow = (cid * NSUB + sid) * chunks + ob * INNER
        pltpu.sync_copy(idx2.at[pl.ds(crow, INNER)], idxb)
        ld, st = {}, {}
        ld[0] = pltpu.async_copy(er3.at[crow], EB[0], sld)
        for j in range(INNER):
            p = j % 2
            if j + 1 < INNER:
                if j >= 1:
                    st[j - 1].wait()
                ld[j + 1] = pltpu.async_copy(er3.at[crow + j + 1], EB[1 - p], sld)
            ld[j].wait()
            st[j] = pltpu.async_copy(EB[p], tab.at[idxb.at[j]], sst, add=True)
        st[INNER - 2].wait()
        st[INNER - 1].wait()
        return carry

    lax.fori_loop(0, outer, body, 0)
    plsc.subcore_barrier()
    mine = pl.ds(sid * RPT_N, RPT_N)
    pltpu.sync_copy(tab.at[mine], o_out.at[cid, mine])


def _sc_seg(er3, idx2):
    mesh = plsc.VectorSubcoreMesh(core_axis_name="c", subcore_axis_name="s", num_cores=NSC, num_subcores=NSUB)
    k = pl.kernel(
        _sc_seg_body,
        out_type=jax.ShapeDtypeStruct((NSC, NP, C), f32),
        mesh=mesh,
        scratch_types=[
            pltpu.VMEM_SHARED((NP, C), f32),
            pltpu.VMEM((W, C), f32),
            pltpu.VMEM((W, C), f32),
            pltpu.VMEM((INNER, W), jnp.int32),
            pltpu.VMEM((64, C), f32),
            pltpu.SemaphoreType.DMA,
            pltpu.SemaphoreType.DMA,
        ],
    )
    return k(er3, idx2)


def _sc_deg_body(idx2, d_out, dtab, obuf, idxb, zbuf, sst):
    cid = lax.axis_index("c")
    sid = lax.axis_index("s")
    _zero_buf(zbuf)
    _fill_table(zbuf, dtab, sid)
    one16 = jnp.ones((16,), f32)

    def ones_body(i, carry):
        for k in range(C // 16):
            obuf[i, pl.ds(16 * k, 16)] = one16
        return carry

    lax.fori_loop(0, W, ones_body, 0)
    plsc.subcore_barrier()

    chunks = NCHUNK // (NSC * NSUB)
    outer = chunks // INNER

    def body(ob, carry):
        crow = (cid * NSUB + sid) * chunks + ob * INNER
        pltpu.sync_copy(idx2.at[pl.ds(crow, INNER)], idxb)
        descs = [pltpu.async_copy(obuf, dtab.at[idxb.at[j]], sst, add=True)
                 for j in range(INNER)]
        for dsc in descs:
            dsc.wait()
        return carry

    lax.fori_loop(0, outer, body, 0)
    plsc.subcore_barrier()
    mine = pl.ds(sid * RPT_N, RPT_N)
    pltpu.sync_copy(dtab.at[mine], d_out.at[cid, mine])


def _sc_deg(idx2):
    mesh = plsc.VectorSubcoreMesh(core_axis_name="c", subcore_axis_name="s", num_cores=NSC, num_subcores=NSUB)
    k = pl.kernel(
        _sc_deg_body,
        out_type=jax.ShapeDtypeStruct((NSC, NP, C), f32),
        mesh=mesh,
        scratch_types=[
            pltpu.VMEM_SHARED((NP, C), f32),
            pltpu.VMEM((W, C), f32),
            pltpu.VMEM((INNER, W), jnp.int32),
            pltpu.VMEM((64, C), f32),
            pltpu.SemaphoreType.DMA,
        ],
    )
    return k(idx2)


def _sc_nbr_body(tbl, nid2, nsw2, o_out, otab, gb0, gb1, nidb, nswb, zbuf,
                 sgt, sst):
    cid = lax.axis_index("c")
    sid = lax.axis_index("s")
    _zero_buf(zbuf)
    _fill_table(zbuf, otab, sid)
    plsc.subcore_barrier()

    chunks = NCHUNK // (NSC * NSUB)
    outer = chunks // INNER
    GB = (gb0, gb1)

    def body(ob, carry):
        crow = (cid * NSUB + sid) * chunks + ob * INNER
        pltpu.sync_copy(nid2.at[pl.ds(crow, INNER)], nidb)
        pltpu.sync_copy(nsw2.at[pl.ds(crow, INNER)], nswb)
        gt, st = {}, {}
        gt[0] = pltpu.async_copy(tbl.at[nswb.at[0]], GB[0], sgt)
        for j in range(INNER):
            p = j % 2
            if j + 1 < INNER:
                if j >= 1:
                    st[j - 1].wait()
                gt[j + 1] = pltpu.async_copy(tbl.at[nswb.at[j + 1]],
                                             GB[1 - p], sgt)
            gt[j].wait()
            st[j] = pltpu.async_copy(GB[p], otab.at[nidb.at[j]], sst, add=True)
        st[INNER - 2].wait()
        st[INNER - 1].wait()
        return carry

    lax.fori_loop(0, outer, body, 0)
    plsc.subcore_barrier()
    mine = pl.ds(sid * RPT_N, RPT_N)
    pltpu.sync_copy(otab.at[mine], o_out.at[cid, mine])


def _sc_nbr(tbl, nid2, nsw2):
    mesh = plsc.VectorSubcoreMesh(core_axis_name="c", subcore_axis_name="s", num_cores=NSC, num_subcores=NSUB)
    k = pl.kernel(
        _sc_nbr_body,
        out_type=jax.ShapeDtypeStruct((NSC, NP, C), f32),
        mesh=mesh,
        scratch_types=[
            pltpu.VMEM_SHARED((NP, C), f32),
            pltpu.VMEM((W, C), f32),
            pltpu.VMEM((W, C), f32),
            pltpu.VMEM((INNER, W), jnp.int32),
            pltpu.VMEM((INNER, W), jnp.int32),
            pltpu.VMEM((64, C), f32),
            pltpu.SemaphoreType.DMA,
            pltpu.SemaphoreType.DMA,
        ],
    )
    return k(tbl, nid2, nsw2)


def _sc_zp_body(f1a, f1b, f2a, f2b, nid2, nsw2, zpa_out, zpb_out,
                a0, a1, a2, b0, b1, b2, nidb, nswb,
                sg1a, sg2a, ssta, sg1b, sg2b, sstb):
    cid = lax.axis_index("c")
    sid = lax.axis_index("s")
    chunks = NCHUNK // (NSC * NSUB)
    outer = chunks // INNER
    A = (a0, a1, a2)
    B = (b0, b1, b2)

    def body(ob, carry):
        crow = (cid * NSUB + sid) * chunks + ob * INNER
        pltpu.sync_copy(nid2.at[pl.ds(crow, INNER)], nidb)
        pltpu.sync_copy(nsw2.at[pl.ds(crow, INNER)], nswb)
        g1, g2, st = {}, {}, {}
        g1[0] = (pltpu.async_copy(f1a.at[nidb.at[0]], A[0], sg1a),
                 pltpu.async_copy(f1b.at[nidb.at[0]], B[0], sg1b))
        g1[1] = (pltpu.async_copy(f1a.at[nidb.at[1]], A[1], sg1a),
                 pltpu.async_copy(f1b.at[nidb.at[1]], B[1], sg1b))
        for j in range(INNER):
            p = j % 3
            if j >= 1:
                da, db = g2[j - 1]
                da.wait()
                sa = pltpu.async_copy(A[(j - 1) % 3], zpa_out.at[crow + j - 1],
                                      ssta)
                db.wait()
                sb = pltpu.async_copy(B[(j - 1) % 3], zpb_out.at[crow + j - 1],
                                      sstb)
                st[j - 1] = (sa, sb)
            if j + 2 < INNER:
                if j >= 1:
                    for dsc in st[j - 1]:
                        dsc.wait()
                g1[j + 2] = (
                    pltpu.async_copy(f1a.at[nidb.at[j + 2]], A[(j + 2) % 3],
                                     sg1a),
                    pltpu.async_copy(f1b.at[nidb.at[j + 2]], B[(j + 2) % 3],
                                     sg1b))
            da, db = g1[j]
            da.wait()
            g2a = pltpu.async_copy(f2a.at[nswb.at[j]], A[p], sg2a, add=True)
            db.wait()
            g2b = pltpu.async_copy(f2b.at[nswb.at[j]], B[p], sg2b, add=True)
            g2[j] = (g2a, g2b)
        da, db = g2[INNER - 1]
        da.wait()
        sa = pltpu.async_copy(A[(INNER - 1) % 3], zpa_out.at[crow + INNER - 1],
                              ssta)
        db.wait()
        sb = pltpu.async_copy(B[(INNER - 1) % 3], zpb_out.at[crow + INNER - 1],
                              sstb)
        for dsc in st[INNER - 3]:
            dsc.wait()
        for dsc in st[INNER - 2]:
            dsc.wait()
        sa.wait()
        sb.wait()
        return carry

    lax.fori_loop(0, outer, body, 0)


def _sc_zp(f1a, f1b, f2a, f2b, nid2, nsw2):
    mesh = plsc.VectorSubcoreMesh(core_axis_name="c", subcore_axis_name="s", num_cores=NSC, num_subcores=NSUB)
    k = pl.kernel(
        _sc_zp_body,
        out_type=(
            jax.ShapeDtypeStruct((NCHUNK, W, C), f32),
            jax.ShapeDtypeStruct((NCHUNK, W, C), f32),
        ),
        mesh=mesh,
        scratch_types=(
            [pltpu.VMEM((W, C), f32)] * 6
            + [pltpu.VMEM((INNER, W), jnp.int32)] * 2
            + [pltpu.SemaphoreType.DMA] * 6
        ),
    )
    return k(f1a, f1b, f2a, f2b, nid2, nsw2)


def _tc_proj_body(sp_ref, tp_ref, x_ref, dp_ref, w7_ref,
                  f1a_ref, f1b_ref, f2a_ref, f2b_ref, w04_ref, s_ref, t_ref):
    i = pl.program_id(0)
    w = w7_ref[...]
    w1234 = w[1] + w[2] + w[3] + w[4]
    w24 = w[2] + w[4]
    w34 = w[3] + w[4]
    w56 = w[5] + w[6]
    s = sp_ref[0] + sp_ref[1]
    t = tp_ref[0] + tp_ref[1]
    x = x_ref[...]
    d = dp_ref[0, :, 0:1] + dp_ref[1, :, 0:1]
    s_ref[...] = s
    t_ref[...] = t
    ds_ = d * s
    dx = d * x
    dot = functools.partial(jnp.dot, preferred_element_type=f32)
    f1 = dot(ds_, w1234) + dot(t, w24) + dot(dx, w56)
    f2 = dot(s, w34) + dot(x, w[6])
    f1a_ref[...] = f1[:, :C]
    f1b_ref[...] = f1[:, C:]
    f2a_ref[...] = f2[:, :C]
    f2b_ref[...] = f2[:, C:]

    @pl.when(i == 0)
    def _():
        w04_ref[...] = w[0] + w[4]


def _tc_proj(sp, tp, x, dp, w7):
    blk = 400
    return pl.pallas_call(
        _tc_proj_body,
        grid=(N // blk,),
        in_specs=[
            pl.BlockSpec((NSC, blk, C), lambda i: (0, i, 0)),
            pl.BlockSpec((NSC, blk, C), lambda i: (0, i, 0)),
            pl.BlockSpec((blk, C), lambda i: (i, 0)),
            pl.BlockSpec((NSC, blk, C), lambda i: (0, i, 0)),
            pl.BlockSpec((7, C, 2 * C), lambda i: (0, 0, 0)),
        ],
        out_specs=[
            pl.BlockSpec((blk, C), lambda i: (i, 0)),
            pl.BlockSpec((blk, C), lambda i: (i, 0)),
            pl.BlockSpec((blk, C), lambda i: (i, 0)),
            pl.BlockSpec((blk, C), lambda i: (i, 0)),
            pl.BlockSpec((C, 2 * C), lambda i: (0, 0)),
            pl.BlockSpec((blk, C), lambda i: (i, 0)),
            pl.BlockSpec((blk, C), lambda i: (i, 0)),
        ],
        out_shape=[
            jax.ShapeDtypeStruct((N, C), f32),
            jax.ShapeDtypeStruct((N, C), f32),
            jax.ShapeDtypeStruct((N, C), f32),
            jax.ShapeDtypeStruct((N, C), f32),
            jax.ShapeDtypeStruct((C, 2 * C), f32),
            jax.ShapeDtypeStruct((N, C), f32),
            jax.ShapeDtypeStruct((N, C), f32),
        ],
    )(sp, tp, x, dp, w7)


def _tc_nodez_body(x_ref, s_ref, t_ref, xsp_ref, ssp_ref, dp_ref, w7_ref,
                   z_ref, st_ref):
    i = pl.program_id(0)
    x = x_ref[...]
    s = s_ref[...]
    t = t_ref[...]
    xs = xsp_ref[0] + xsp_ref[1]
    ss = ssp_ref[0] + ssp_ref[1]
    d = dp_ref[0, :, 0:1] + dp_ref[1, :, 0:1]
    w = w7_ref[...]
    ds_ = d * s
    d2s = d * ds_
    dt = d * t
    dx = d * x
    d2x = d * dx
    dot = functools.partial(jnp.dot, preferred_element_type=f32)
    z = (dot(x, w[0]) + dot(d2s, w[1] + w[2] + w[3] + w[4])
         + dot(dt, w[2] + w[4]) + dot(ss, w[3] + w[4]) + dot(s, w[4])
         + dot(d2x, w[5] + w[6]) + dot(xs, w[6]))
    z_ref[...] = z

    @pl.when(i == 0)
    def _():
        st_ref[...] = jnp.zeros_like(st_ref)

    st_ref[0:1, :] += jnp.sum(z, axis=0, keepdims=True)
    st_ref[1:2, :] += jnp.sum(z * z, axis=0, keepdims=True)


def _tc_nodez(x, s, t, xsp, ssp, dp, w7):
    blk = 2000
    return pl.pallas_call(
        _tc_nodez_body,
        grid=(N // blk,),
        in_specs=[
            pl.BlockSpec((blk, C), lambda i: (i, 0)),
            pl.BlockSpec((blk, C), lambda i: (i, 0)),
            pl.BlockSpec((blk, C), lambda i: (i, 0)),
            pl.BlockSpec((NSC, blk, C), lambda i: (0, i, 0)),
            pl.BlockSpec((NSC, blk, C), lambda i: (0, i, 0)),
            pl.BlockSpec((NSC, blk, C), lambda i: (0, i, 0)),
            pl.BlockSpec((7, C, 2 * C), lambda i: (0, 0, 0)),
        ],
        out_specs=[
            pl.BlockSpec((blk, 2 * C), lambda i: (i, 0)),
            pl.BlockSpec((8, 2 * C), lambda i: (0, 0)),
        ],
        out_shape=[
            jax.ShapeDtypeStruct((N, 2 * C), f32),
            jax.ShapeDtypeStruct((8, 2 * C), f32),
        ],
    )(x, s, t, xsp, ssp, dp, w7)


def _tc_norm_mm_body(nrows, y_ref, st_ref, g_ref, b_ref, w2_ref, y2_ref, st2_ref):
    i = pl.program_id(0)
    m = st_ref[0:1, :] * (1.0 / nrows)
    v = st_ref[1:2, :] * (1.0 / nrows) - m * m
    h = jnp.maximum(
        g_ref[...] * (y_ref[...] - m) * lax.rsqrt(v + 1e-5) + b_ref[...], 0.0)
    y2 = jnp.dot(h, w2_ref[...], preferred_element_type=f32)
    y2_ref[...] = y2

    @pl.when(i == 0)
    def _():
        st2_ref[...] = jnp.zeros_like(st2_ref)

    st2_ref[0:1, :] += jnp.sum(y2, axis=0, keepdims=True)
    st2_ref[1:2, :] += jnp.sum(y2 * y2, axis=0, keepdims=True)


def _tc_norm_mm(y, st, g, b, w2, blk):
    nrows, cin = y.shape
    cout = w2.shape[1]
    return pl.pallas_call(
        functools.partial(_tc_norm_mm_body, nrows),
        grid=(nrows // blk,),
        in_specs=[
            pl.BlockSpec((blk, cin), lambda i: (i, 0)),
            pl.BlockSpec((8, cin), lambda i: (0, 0)),
            pl.BlockSpec((1, cin), lambda i: (0, 0)),
            pl.BlockSpec((1, cin), lambda i: (0, 0)),
            pl.BlockSpec((cin, cout), lambda i: (0, 0)),
        ],
        out_specs=[
            pl.BlockSpec((blk, cout), lambda i: (i, 0)),
            pl.BlockSpec((8, cout), lambda i: (0, 0)),
        ],
        out_shape=[
            jax.ShapeDtypeStruct((nrows, cout), f32),
            jax.ShapeDtypeStruct((8, cout), f32),
        ],
    )(y, st, g.reshape(1, -1), b.reshape(1, -1), w2)


def _tc_norm_body(nrows, y_ref, st_ref, g_ref, b_ref, out_ref):
    m = st_ref[0:1, :] * (1.0 / nrows)
    v = st_ref[1:2, :] * (1.0 / nrows) - m * m
    out_ref[...] = jnp.maximum(
        g_ref[...] * (y_ref[...] - m) * lax.rsqrt(v + 1e-5) + b_ref[...], 0.0)


def _tc_norm(y, st, g, b, blk):
    nrows, cin = y.shape
    return pl.pallas_call(
        functools.partial(_tc_norm_body, nrows),
        grid=(nrows // blk,),
        in_specs=[
            pl.BlockSpec((blk, cin), lambda i: (i, 0)),
            pl.BlockSpec((8, cin), lambda i: (0, 0)),
            pl.BlockSpec((1, cin), lambda i: (0, 0)),
            pl.BlockSpec((1, cin), lambda i: (0, 0)),
        ],
        out_specs=pl.BlockSpec((blk, cin), lambda i: (i, 0)),
        out_shape=jax.ShapeDtypeStruct((nrows, cin), f32),
    )(y, st, g.reshape(1, -1), b.reshape(1, -1))


def _tc_estats_body(er_ref, zpa_ref, zpb_ref, w04_ref, out_ref):
    i = pl.program_id(0)

    @pl.when(i == 0)
    def _():
        out_ref[...] = jnp.zeros_like(out_ref)

    zp = jnp.concatenate([zpa_ref[...], zpb_ref[...]], axis=-1)
    z = (jnp.dot(er_ref[...], w04_ref[...], preferred_element_type=f32)
         + zp)
    out_ref[0:1, :] += jnp.sum(z, axis=0, keepdims=True)
    out_ref[1:2, :] += jnp.sum(z * z, axis=0, keepdims=True)


def _tc_estats(er, zpa, zpb, w04, blk):
    return pl.pallas_call(
        _tc_estats_body,
        grid=(R // blk,),
        in_specs=[
            pl.BlockSpec((blk, C), lambda i: (i, 0)),
            pl.BlockSpec((blk, C), lambda i: (i, 0)),
            pl.BlockSpec((blk, C), lambda i: (i, 0)),
            pl.BlockSpec((C, 2 * C), lambda i: (0, 0)),
        ],
        out_specs=pl.BlockSpec((8, 2 * C), lambda i: (0, 0)),
        out_shape=jax.ShapeDtypeStruct((8, 2 * C), f32),
    )(er, zpa, zpb, w04)


def _tc_elayer_body(er_ref, zpa_ref, zpb_ref, w04_ref, st1_ref, ge1_ref,
                    be1_ref, we2_ref, h2_ref, st2_ref):
    i = pl.program_id(0)
    zp = jnp.concatenate([zpa_ref[...], zpb_ref[...]], axis=-1)
    z = (jnp.dot(er_ref[...], w04_ref[...], preferred_element_type=f32)
         + zp)
    m = st1_ref[0:1, :] * (1.0 / R)
    v = st1_ref[1:2, :] * (1.0 / R) - m * m
    h = jnp.maximum(ge1_ref[...] * (z - m) * lax.rsqrt(v + 1e-5) + be1_ref[...], 0.0)
    h2 = jnp.dot(h, we2_ref[...], preferred_element_type=f32)
    h2_ref[...] = h2

    @pl.when(i == 0)
    def _():
        st2_ref[...] = jnp.zeros_like(st2_ref)

    st2_ref[0:1, :] += jnp.sum(h2, axis=0, keepdims=True)
    st2_ref[1:2, :] += jnp.sum(h2 * h2, axis=0, keepdims=True)


def _tc_elayer(er, zpa, zpb, w04, st1, ge1, be1, we2, blk):
    return pl.pallas_call(
        _tc_elayer_body,
        grid=(R // blk,),
        in_specs=[
            pl.BlockSpec((blk, C), lambda i: (i, 0)),
            pl.BlockSpec((blk, C), lambda i: (i, 0)),
            pl.BlockSpec((blk, C), lambda i: (i, 0)),
            pl.BlockSpec((C, 2 * C), lambda i: (0, 0)),
            pl.BlockSpec((8, 2 * C), lambda i: (0, 0)),
            pl.BlockSpec((1, 2 * C), lambda i: (0, 0)),
            pl.BlockSpec((1, 2 * C), lambda i: (0, 0)),
            pl.BlockSpec((2 * C, C), lambda i: (0, 0)),
        ],
        out_specs=[
            pl.BlockSpec((blk, C), lambda i: (i, 0)),
            pl.BlockSpec((8, C), lambda i: (0, 0)),
        ],
        out_shape=[
            jax.ShapeDtypeStruct((R, C), f32),
            jax.ShapeDtypeStruct((8, C), f32),
        ],
    )(er, zpa, zpb, w04, st1, ge1.reshape(1, -1), be1.reshape(1, -1), we2)


def kernel(node_rep, edge_rep, edge_index, Wn1, gn1, bn1, Wn2, gn2, bn2,
           We1, ge1, be1, We2, ge2, be2):
    u = edge_index[0]
    v = edge_index[1]
    nid2 = jnp.stack([u, v], axis=1).reshape(NCHUNK, W)
    nsw2 = jnp.stack([v, u], axis=1).reshape(NCHUNK, W)
    er3 = edge_rep.reshape(NCHUNK, W, C)
    w7e = We1.reshape(7, C, 2 * C)
    w7n = Wn1.reshape(7, C, 2 * C)

    dp = _sc_deg(nid2)
    xsp = _sc_nbr(node_rep, nid2, nsw2)
    sp = _sc_seg(er3, nid2)
    tp = _sc_seg(er3, nsw2)

    dpn = dp[:, :N]
    f1a, f1b, f2a, f2b, w04, s_sum, t_sum = _tc_proj(sp[:, :N], tp[:, :N],
                                                     node_rep, dpn, w7e)
    ssp = _sc_nbr(s_sum, nid2, nsw2)
    zpa3, zpb3 = _sc_zp(f1a, f1b, f2a, f2b, nid2, nsw2)
    zpa = zpa3.reshape(R, C)
    zpb = zpb3.reshape(R, C)

    zn, stn1 = _tc_nodez(node_rep, s_sum, t_sum, xsp[:, :N], ssp[:, :N],
                         dpn, w7n)
    zn2, stn2 = _tc_norm_mm(zn, stn1, gn1, bn1, Wn2, 2000)
    node_out = _tc_norm(zn2, stn2, gn2, bn2, 2000)

    blk = 2000
    st1 = _tc_estats(edge_rep, zpa, zpb, w04, blk)
    h2, st2 = _tc_elayer(edge_rep, zpa, zpb, w04, st1, ge1, be1, We2, blk)
    edge_out = _tc_norm(h2, st2, ge2, be2, blk)
    return (node_out, edge_out)
